# baseline (speedup 1.0000x reference)
import functools

import jax
import jax.numpy as jnp
from jax import lax
from jax.experimental import pallas as pl
from jax.experimental.pallas import tpu as pltpu
from jax.experimental.pallas import tpu_sc as plsc

N = 10000
E = 320000
D = 128
NE = 4
STEPS = 8

NW = 32
EPW = E // NW
K = 128
CHUNKS = 80
EPW_PAD = K * CHUNKS
GRP = 8
NGRP = CHUNKS // GRP
NPAIRS = CHUNKS // 2
N_ACC = 10112
ZR = 128
RPT = N_ACC // 16

NB = 2000
GRID = N // NB


def _y_body(h_ref, wcat_ref, bcat_ref, y_ref):
    ycat = jnp.dot(h_ref[...], wcat_ref[...],
                   preferred_element_type=jnp.float32) + bcat_ref[...]
    for t in range(NE):
        y_ref[t] = ycat[:, t * D:(t + 1) * D]


_y_call = pl.pallas_call(
    _y_body,
    grid=(GRID,),
    in_specs=[
        pl.BlockSpec((NB, D), lambda i: (i, 0)),
        pl.BlockSpec((D, NE * D), lambda i: (0, 0)),
        pl.BlockSpec((1, NE * D), lambda i: (0, 0)),
    ],
    out_specs=pl.BlockSpec((NE, NB, D), lambda i: (0, i, 0)),
    out_shape=jax.ShapeDtypeStruct((NE, N, D), jnp.float32),
)


def _gru(a2_ref, h_ref, wih_ref, whh_ref, bih_ref, bhh_ref):
    a = a2_ref[0] + a2_ref[1]
    h = h_ref[...]
    gi = jnp.dot(a, wih_ref[...], preferred_element_type=jnp.float32) + bih_ref[...]
    gh = jnp.dot(h, whh_ref[...], preferred_element_type=jnp.float32) + bhh_ref[...]
    r = jax.nn.sigmoid(gi[:, :D] + gh[:, :D])
    z = jax.nn.sigmoid(gi[:, D:2 * D] + gh[:, D:2 * D])
    n = jnp.tanh(gi[:, 2 * D:] + r * gh[:, 2 * D:])
    return (1.0 - z) * n + z * h


def _step_body(a2_ref, h_ref, wih_ref, whh_ref, bih_ref, bhh_ref,
               wcat_ref, bcat_ref, h_out_ref, y_ref):
    hn = _gru(a2_ref, h_ref, wih_ref, whh_ref, bih_ref, bhh_ref)
    h_out_ref[...] = hn
    ycat = jnp.dot(hn, wcat_ref[...],
                   preferred_element_type=jnp.float32) + bcat_ref[...]
    for t in range(NE):
        y_ref[t] = ycat[:, t * D:(t + 1) * D]


_step_call = pl.pallas_call(
    _step_body,
    grid=(GRID,),
    in_specs=[
        pl.BlockSpec((2, NB, D), lambda i: (0, i, 0)),
        pl.BlockSpec((NB, D), lambda i: (i, 0)),
        pl.BlockSpec((D, 3 * D), lambda i: (0, 0)),
        pl.BlockSpec((D, 3 * D), lambda i: (0, 0)),
        pl.BlockSpec((1, 3 * D), lambda i: (0, 0)),
        pl.BlockSpec((1, 3 * D), lambda i: (0, 0)),
        pl.BlockSpec((D, NE * D), lambda i: (0, 0)),
        pl.BlockSpec((1, NE * D), lambda i: (0, 0)),
    ],
    out_specs=[
        pl.BlockSpec((NB, D), lambda i: (i, 0)),
        pl.BlockSpec((NE, NB, D), lambda i: (0, i, 0)),
    ],
    out_shape=[
        jax.ShapeDtypeStruct((N, D), jnp.float32),
        jax.ShapeDtypeStruct((NE, N, D), jnp.float32),
    ],
)


def _final_body(a2_ref, h_ref, wih_ref, whh_ref, bih_ref, bhh_ref,
                wcls_ref, bcls_ref, out_ref, acc_ref):
    i = pl.program_id(0)
    hn = _gru(a2_ref, h_ref, wih_ref, whh_ref, bih_ref, bhh_ref)

    @pl.when(i == 0)
    def _():
        acc_ref[...] = jnp.zeros_like(acc_ref)

    acc_ref[...] += jnp.sum(hn, axis=0, keepdims=True)

    @pl.when(i == GRID - 1)
    def _():
        logit = jnp.sum(acc_ref[...] * wcls_ref[...], axis=1,
                        keepdims=True) + bcls_ref[...]
        out_ref[...] = jax.nn.sigmoid(logit)


_final_call = pl.pallas_call(
    _final_body,
    grid=(GRID,),
    in_specs=[
        pl.BlockSpec((2, NB, D), lambda i: (0, i, 0)),
        pl.BlockSpec((NB, D), lambda i: (i, 0)),
        pl.BlockSpec((D, 3 * D), lambda i: (0, 0)),
        pl.BlockSpec((D, 3 * D), lambda i: (0, 0)),
        pl.BlockSpec((1, 3 * D), lambda i: (0, 0)),
        pl.BlockSpec((1, 3 * D), lambda i: (0, 0)),
        pl.BlockSpec((1, D), lambda i: (0, 0)),
        pl.BlockSpec((1, 1), lambda i: (0, 0)),
    ],
    out_specs=pl.BlockSpec((1, 1), lambda i: (0, 0)),
    out_shape=jax.ShapeDtypeStruct((1, 1), jnp.float32),
    scratch_shapes=[pltpu.VMEM((1, D), jnp.float32)],
)


@functools.cache
def _get_sc_scatter():
    mesh = plsc.VectorSubcoreMesh(core_axis_name="c", subcore_axis_name="s")

    @functools.partial(
        pl.kernel,
        out_type=jax.ShapeDtypeStruct((2 * N, D), jnp.float32),
        mesh=mesh,
        scratch_types=[
            pltpu.VMEM((2, GRP, K), jnp.int32),
            pltpu.VMEM((2, GRP, K), jnp.int32),
            pltpu.VMEM((2, K, D), jnp.float32),
            pltpu.VMEM_SHARED((N_ACC, D), jnp.float32),
            pltpu.SemaphoreType.DMA,
            pltpu.SemaphoreType.DMA,
            pltpu.SemaphoreType.DMA,
            pltpu.SemaphoreType.DMA,
            pltpu.SemaphoreType.DMA,
            pltpu.SemaphoreType.DMA,
        ],
    )
    def _sc_scatter(gidx_hbm, dst_hbm, zeros_hbm, y_hbm, out_hbm,
                    gidx_v, dst_v, rows_v, acc_s,
                    semA, semB, semSA, semSB, semIg, semId):
        cid = lax.axis_index("c")
        sid = lax.axis_index("s")
        w = cid * 16 + sid

        pltpu.sync_copy(gidx_hbm.at[w, pl.ds(0, GRP)], gidx_v.at[0])
        pltpu.sync_copy(dst_hbm.at[w, pl.ds(0, GRP)], dst_v.at[0])
        pltpu.make_async_copy(y_hbm.at[gidx_v.at[0, 0]], rows_v.at[0],
                              semA).start()

        pltpu.sync_copy(zeros_hbm, rows_v.at[1])
        for i in range(RPT // K):
            pltpu.sync_copy(rows_v.at[1], acc_s.at[pl.ds(sid * RPT + i * K, K)])
        pltpu.sync_copy(rows_v.at[1, pl.ds(0, RPT % K)],
                        acc_s.at[pl.ds(sid * RPT + (RPT // K) * K, RPT % K)])
        plsc.subcore_barrier()

        def body(p, carry):
            j0 = 2 * p
            g = j0 // GRP
            b = lax.rem(g, 2)
            r0 = j0 - g * GRP
            r1 = r0 + 1

            @pl.when(jnp.logical_and(r0 == 0, g + 1 < NGRP))
            def _():
                gb = lax.rem(g + 1, 2)
                pltpu.make_async_copy(
                    gidx_hbm.at[w, pl.ds((g + 1) * GRP, GRP)],
                    gidx_v.at[gb], semIg).start()
                pltpu.make_async_copy(
                    dst_hbm.at[w, pl.ds((g + 1) * GRP, GRP)],
                    dst_v.at[gb], semId).start()

            pltpu.make_async_copy(y_hbm.at[gidx_v.at[b, r0]], rows_v.at[0],
                                  semA).wait()
            pltpu.make_async_copy(y_hbm.at[gidx_v.at[b, r1]], rows_v.at[1],
                                  semB).start()
            pltpu.sync_copy(rows_v.at[0], acc_s.at[dst_v.at[b, r0]], add=True)

            @pl.when(jnp.logical_and(r1 == GRP - 1, g + 1 < NGRP))
            def _():
                gb = lax.rem(g + 1, 2)
                pltpu.make_async_copy(
                    gidx_hbm.at[w, pl.ds((g + 1) * GRP, GRP)],
                    gidx_v.at[gb], semIg).wait()
                pltpu.make_async_copy(
                    dst_hbm.at[w, pl.ds((g + 1) * GRP, GRP)],
                    dst_v.at[gb], semId).wait()

            @pl.when(p + 1 < NPAIRS)
            def _():
                jn = j0 + 2
                gn = jn // GRP
                pltpu.make_async_copy(
                    y_hbm.at[gidx_v.at[lax.rem(gn, 2), jn - gn * GRP]],
                    rows_v.at[0], semA).start()

            pltpu.make_async_copy(y_hbm.at[gidx_v.at[b, r1]], rows_v.at[1],
                                  semB).wait()
            pltpu.sync_copy(rows_v.at[1], acc_s.at[dst_v.at[b, r1]], add=True)
            return carry

        lax.fori_loop(0, NPAIRS, body, 0)
        plsc.subcore_barrier()

        for i in range(4):
            r0 = sid * 624 + i * K
            pltpu.sync_copy(acc_s.at[pl.ds(r0, K)], rows_v.at[0])
            pltpu.sync_copy(rows_v.at[0], out_hbm.at[pl.ds(cid * N + r0, K)])
        r0 = sid * 624 + 512
        pltpu.sync_copy(acc_s.at[pl.ds(r0, 112)], rows_v.at[0, pl.ds(0, 112)])
        pltpu.sync_copy(rows_v.at[0, pl.ds(0, 112)],
                        out_hbm.at[pl.ds(cid * N + r0, 112)])

        @pl.when(sid == 0)
        def _():
            pltpu.sync_copy(acc_s.at[pl.ds(9984, 16)],
                            rows_v.at[0, pl.ds(0, 16)])
            pltpu.sync_copy(rows_v.at[0, pl.ds(0, 16)],
                            out_hbm.at[pl.ds(cid * N + 9984, 16)])

    return _sc_scatter


def kernel(x, W_lin, b_lin, W_ih, W_hh, b_ih, b_hh, W_cls, b_cls,
           edge_index, edge_types):
    src = edge_index[0]
    dst = edge_index[1]
    gidx = edge_types * N + src
    perm = jnp.argsort(dst)
    gidx = gidx[perm]
    dst = dst[perm]
    g3 = jnp.pad(gidx.reshape(NW, EPW), ((0, 0), (0, EPW_PAD - EPW)),
                 constant_values=0).reshape(NW, CHUNKS, K)
    d3 = jnp.pad(dst.reshape(NW, EPW), ((0, 0), (0, EPW_PAD - EPW)),
                 constant_values=N).reshape(NW, CHUNKS, K)
    zeros = jnp.zeros((ZR, D), jnp.float32)

    wih_t = W_ih.T
    whh_t = W_hh.T
    bih2 = b_ih.reshape(1, 3 * D)
    bhh2 = b_hh.reshape(1, 3 * D)
    wcat = jnp.concatenate([W_lin[t].T for t in range(NE)], axis=1)
    bcat = b_lin.reshape(1, NE * D)

    sc_scatter = _get_sc_scatter()
    h = x
    y = _y_call(h, wcat, bcat)
    for step in range(STEPS):
        a2 = sc_scatter(g3, d3, zeros, y.reshape(NE * N, D))
        a2 = a2.reshape(2, N, D)
        if step < STEPS - 1:
            h, y = _step_call(a2, h, wih_t, whh_t, bih2, bhh2, wcat, bcat)
        else:
            out = _final_call(a2, h, wih_t, whh_t, bih2, bhh2,
                              W_cls, b_cls.reshape(1, 1))
    return out.reshape(1)


# speedup vs baseline: 1.1329x; 1.1329x over previous
import functools

import jax
import jax.numpy as jnp
from jax import lax
from jax.experimental import pallas as pl
from jax.experimental.pallas import tpu as pltpu
from jax.experimental.pallas import tpu_sc as plsc

N = 10000
E = 320000
D = 128
NE = 4
STEPS = 8

NW = 32
EPW = E // NW
K = 128
CHUNKS = 80
EPW_PAD = K * CHUNKS
GRP = 8
NGRP = CHUNKS // GRP
NPAIRS = CHUNKS // 2
N_ACC = 10112
ZR = 128
RPT = N_ACC // 16

NB = 2000
GRID = N // NB


def _y_body(h_ref, wcat_ref, bcat_ref, y_ref):
    ycat = jnp.dot(h_ref[...], wcat_ref[...],
                   preferred_element_type=jnp.float32) + bcat_ref[...]
    for t in range(NE):
        y_ref[t] = ycat[:, t * D:(t + 1) * D]


_y_call = pl.pallas_call(
    _y_body,
    grid=(GRID,),
    in_specs=[
        pl.BlockSpec((NB, D), lambda i: (i, 0)),
        pl.BlockSpec((D, NE * D), lambda i: (0, 0)),
        pl.BlockSpec((1, NE * D), lambda i: (0, 0)),
    ],
    out_specs=pl.BlockSpec((NE, NB, D), lambda i: (0, i, 0)),
    out_shape=jax.ShapeDtypeStruct((NE, N, D), jnp.float32),
)


def _gru(a2_ref, h_ref, wih_ref, whh_ref, bih_ref, bhh_ref):
    a = a2_ref[0] + a2_ref[1]
    h = h_ref[...]
    gi = jnp.dot(a, wih_ref[...], preferred_element_type=jnp.float32) + bih_ref[...]
    gh = jnp.dot(h, whh_ref[...], preferred_element_type=jnp.float32) + bhh_ref[...]
    r = jax.nn.sigmoid(gi[:, :D] + gh[:, :D])
    z = jax.nn.sigmoid(gi[:, D:2 * D] + gh[:, D:2 * D])
    n = jnp.tanh(gi[:, 2 * D:] + r * gh[:, 2 * D:])
    return (1.0 - z) * n + z * h


def _step_body(a2_ref, h_ref, wih_ref, whh_ref, bih_ref, bhh_ref,
               wcat_ref, bcat_ref, h_out_ref, y_ref):
    hn = _gru(a2_ref, h_ref, wih_ref, whh_ref, bih_ref, bhh_ref)
    h_out_ref[...] = hn
    ycat = jnp.dot(hn, wcat_ref[...],
                   preferred_element_type=jnp.float32) + bcat_ref[...]
    for t in range(NE):
        y_ref[t] = ycat[:, t * D:(t + 1) * D]


_step_call = pl.pallas_call(
    _step_body,
    grid=(GRID,),
    in_specs=[
        pl.BlockSpec((2, NB, D), lambda i: (0, i, 0)),
        pl.BlockSpec((NB, D), lambda i: (i, 0)),
        pl.BlockSpec((D, 3 * D), lambda i: (0, 0)),
        pl.BlockSpec((D, 3 * D), lambda i: (0, 0)),
        pl.BlockSpec((1, 3 * D), lambda i: (0, 0)),
        pl.BlockSpec((1, 3 * D), lambda i: (0, 0)),
        pl.BlockSpec((D, NE * D), lambda i: (0, 0)),
        pl.BlockSpec((1, NE * D), lambda i: (0, 0)),
    ],
    out_specs=[
        pl.BlockSpec((NB, D), lambda i: (i, 0)),
        pl.BlockSpec((NE, NB, D), lambda i: (0, i, 0)),
    ],
    out_shape=[
        jax.ShapeDtypeStruct((N, D), jnp.float32),
        jax.ShapeDtypeStruct((NE, N, D), jnp.float32),
    ],
)


def _final_body(a2_ref, h_ref, wih_ref, whh_ref, bih_ref, bhh_ref,
                wcls_ref, bcls_ref, out_ref, acc_ref):
    i = pl.program_id(0)
    hn = _gru(a2_ref, h_ref, wih_ref, whh_ref, bih_ref, bhh_ref)

    @pl.when(i == 0)
    def _():
        acc_ref[...] = jnp.zeros_like(acc_ref)

    acc_ref[...] += jnp.sum(hn, axis=0, keepdims=True)

    @pl.when(i == GRID - 1)
    def _():
        logit = jnp.sum(acc_ref[...] * wcls_ref[...], axis=1,
                        keepdims=True) + bcls_ref[...]
        out_ref[...] = jax.nn.sigmoid(logit)


_final_call = pl.pallas_call(
    _final_body,
    grid=(GRID,),
    in_specs=[
        pl.BlockSpec((2, NB, D), lambda i: (0, i, 0)),
        pl.BlockSpec((NB, D), lambda i: (i, 0)),
        pl.BlockSpec((D, 3 * D), lambda i: (0, 0)),
        pl.BlockSpec((D, 3 * D), lambda i: (0, 0)),
        pl.BlockSpec((1, 3 * D), lambda i: (0, 0)),
        pl.BlockSpec((1, 3 * D), lambda i: (0, 0)),
        pl.BlockSpec((1, D), lambda i: (0, 0)),
        pl.BlockSpec((1, 1), lambda i: (0, 0)),
    ],
    out_specs=pl.BlockSpec((1, 1), lambda i: (0, 0)),
    out_shape=jax.ShapeDtypeStruct((1, 1), jnp.float32),
    scratch_shapes=[pltpu.VMEM((1, D), jnp.float32)],
)


@functools.cache
def _get_sc_scatter():
    mesh = plsc.VectorSubcoreMesh(core_axis_name="c", subcore_axis_name="s")

    @functools.partial(
        pl.kernel,
        out_type=jax.ShapeDtypeStruct((2 * N, D), jnp.float32),
        mesh=mesh,
        scratch_types=[
            pltpu.VMEM((2, GRP, K), jnp.int32),
            pltpu.VMEM((2, GRP, K), jnp.int32),
            pltpu.VMEM((2, K, D), jnp.float32),
            pltpu.VMEM_SHARED((N_ACC, D), jnp.float32),
            pltpu.SemaphoreType.DMA,
            pltpu.SemaphoreType.DMA,
            pltpu.SemaphoreType.DMA,
            pltpu.SemaphoreType.DMA,
            pltpu.SemaphoreType.DMA,
            pltpu.SemaphoreType.DMA,
        ],
    )
    def _sc_scatter(gidx_hbm, dst_hbm, zeros_hbm, y_hbm, out_hbm,
                    gidx_v, dst_v, rows_v, acc_s,
                    semA, semB, semSA, semSB, semIg, semId):
        cid = lax.axis_index("c")
        sid = lax.axis_index("s")
        w = cid * 16 + sid

        pltpu.sync_copy(gidx_hbm.at[w, pl.ds(0, GRP)], gidx_v.at[0])
        pltpu.sync_copy(dst_hbm.at[w, pl.ds(0, GRP)], dst_v.at[0])
        pltpu.make_async_copy(y_hbm.at[gidx_v.at[0, 0]], rows_v.at[0],
                              semA).start()

        pltpu.sync_copy(zeros_hbm, rows_v.at[1])
        for i in range(RPT // K):
            pltpu.sync_copy(rows_v.at[1], acc_s.at[pl.ds(sid * RPT + i * K, K)])
        pltpu.sync_copy(rows_v.at[1, pl.ds(0, RPT % K)],
                        acc_s.at[pl.ds(sid * RPT + (RPT // K) * K, RPT % K)])
        plsc.subcore_barrier()

        def body(p, carry):
            j0 = 2 * p
            g = j0 // GRP
            b = lax.rem(g, 2)
            r0 = j0 - g * GRP
            r1 = r0 + 1

            @pl.when(jnp.logical_and(r0 == 0, g + 1 < NGRP))
            def _():
                gb = lax.rem(g + 1, 2)
                pltpu.make_async_copy(
                    gidx_hbm.at[w, pl.ds((g + 1) * GRP, GRP)],
                    gidx_v.at[gb], semIg).start()
                pltpu.make_async_copy(
                    dst_hbm.at[w, pl.ds((g + 1) * GRP, GRP)],
                    dst_v.at[gb], semId).start()

            pltpu.make_async_copy(y_hbm.at[gidx_v.at[b, r0]], rows_v.at[0],
                                  semA).wait()
            pltpu.make_async_copy(y_hbm.at[gidx_v.at[b, r1]], rows_v.at[1],
                                  semB).start()
            pltpu.sync_copy(rows_v.at[0], acc_s.at[dst_v.at[b, r0]], add=True)

            @pl.when(jnp.logical_and(r1 == GRP - 1, g + 1 < NGRP))
            def _():
                gb = lax.rem(g + 1, 2)
                pltpu.make_async_copy(
                    gidx_hbm.at[w, pl.ds((g + 1) * GRP, GRP)],
                    gidx_v.at[gb], semIg).wait()
                pltpu.make_async_copy(
                    dst_hbm.at[w, pl.ds((g + 1) * GRP, GRP)],
                    dst_v.at[gb], semId).wait()

            @pl.when(p + 1 < NPAIRS)
            def _():
                jn = j0 + 2
                gn = jn // GRP
                pltpu.make_async_copy(
                    y_hbm.at[gidx_v.at[lax.rem(gn, 2), jn - gn * GRP]],
                    rows_v.at[0], semA).start()

            pltpu.make_async_copy(y_hbm.at[gidx_v.at[b, r1]], rows_v.at[1],
                                  semB).wait()
            pltpu.sync_copy(rows_v.at[1], acc_s.at[dst_v.at[b, r1]], add=True)
            return carry

        lax.fori_loop(0, NPAIRS, body, 0)
        plsc.subcore_barrier()

        for i in range(4):
            r0 = sid * 624 + i * K
            pltpu.sync_copy(acc_s.at[pl.ds(r0, K)], rows_v.at[0])
            pltpu.sync_copy(rows_v.at[0], out_hbm.at[pl.ds(cid * N + r0, K)])
        r0 = sid * 624 + 512
        pltpu.sync_copy(acc_s.at[pl.ds(r0, 112)], rows_v.at[0, pl.ds(0, 112)])
        pltpu.sync_copy(rows_v.at[0, pl.ds(0, 112)],
                        out_hbm.at[pl.ds(cid * N + r0, 112)])

        @pl.when(sid == 0)
        def _():
            pltpu.sync_copy(acc_s.at[pl.ds(9984, 16)],
                            rows_v.at[0, pl.ds(0, 16)])
            pltpu.sync_copy(rows_v.at[0, pl.ds(0, 16)],
                            out_hbm.at[pl.ds(cid * N + 9984, 16)])

    return _sc_scatter


def kernel(x, W_lin, b_lin, W_ih, W_hh, b_ih, b_hh, W_cls, b_cls,
           edge_index, edge_types):
    src = edge_index[0]
    dst = edge_index[1]
    gidx = edge_types * N + src
    g3 = jnp.pad(gidx.reshape(NW, EPW), ((0, 0), (0, EPW_PAD - EPW)),
                 constant_values=0).reshape(NW, CHUNKS, K)
    d3 = jnp.pad(dst.reshape(NW, EPW), ((0, 0), (0, EPW_PAD - EPW)),
                 constant_values=N).reshape(NW, CHUNKS, K)
    zeros = jnp.zeros((ZR, D), jnp.float32)

    wih_t = W_ih.T
    whh_t = W_hh.T
    bih2 = b_ih.reshape(1, 3 * D)
    bhh2 = b_hh.reshape(1, 3 * D)
    wcat = jnp.concatenate([W_lin[t].T for t in range(NE)], axis=1)
    bcat = b_lin.reshape(1, NE * D)

    sc_scatter = _get_sc_scatter()
    h = x
    y = _y_call(h, wcat, bcat)
    for step in range(STEPS):
        a2 = sc_scatter(g3, d3, zeros, y.reshape(NE * N, D))
        a2 = a2.reshape(2, N, D)
        if step < STEPS - 1:
            h, y = _step_call(a2, h, wih_t, whh_t, bih2, bhh2, wcat, bcat)
        else:
            out = _final_call(a2, h, wih_t, whh_t, bih2, bhh2,
                              W_cls, b_cls.reshape(1, 1))
    return out.reshape(1)
